# baseline (speedup 1.0000x reference)
import functools

import jax
import jax.numpy as jnp
from jax import lax
from jax.experimental import pallas as pl
from jax.experimental.pallas import tpu as pltpu
from jax.experimental.pallas import tpu_sc as plsc

D = 16
NV = 1000000
B = 16384
NC = 2
NS = 16
NW = NC * NS
BPW = B // NW
CH = 128
NCH = BPW * D // CH

_mesh = plsc.VectorSubcoreMesh(core_axis_name="c", subcore_axis_name="s")


@functools.partial(
    pl.kernel,
    out_type=jax.ShapeDtypeStruct((B,), jnp.float32),
    mesh=_mesh,
    compiler_params=pltpu.CompilerParams(
        needs_layout_passes=False, use_tc_tiling_on_sc=False),
    scratch_types=[
        pltpu.VMEM((BPW,), jnp.int32),
        pltpu.VMEM((BPW,), jnp.int32),
        pltpu.VMEM((BPW * D,), jnp.int32),
        pltpu.VMEM((BPW * D,), jnp.int32),
        pltpu.VMEM((BPW * D,), jnp.float32),
        pltpu.VMEM((BPW * D,), jnp.float32),
        pltpu.VMEM((BPW,), jnp.float32),
        pltpu.SemaphoreType.DMA,
    ],
)
def _skipgram_sc(ci_hbm, xi_hbm, cwf_hbm, xwf_hbm, out_hbm,
                 ci_v, xi_v, co_v, xo_v, v_f, u_f, o_v, sem):
    wid = lax.axis_index("s") * NC + lax.axis_index("c")
    base = wid * BPW

    pltpu.sync_copy(ci_hbm.at[pl.ds(base, BPW)], ci_v)
    pltpu.sync_copy(xi_hbm.at[pl.ds(base, BPW)], xi_v)

    def off_body(k, carry):
        civ = ci_v[pl.ds(k * 16, 16)]
        xiv = xi_v[pl.ds(k * 16, 16)]
        for d in range(D):
            sl = pl.ds(d * BPW + k * 16, 16)
            co_v[sl] = civ + d * NV
            xo_v[sl] = xiv + d * NV
        return carry

    lax.fori_loop(0, BPW // 16, off_body, 0)

    copies = []
    for c in range(NCH):
        sl = pl.ds(c * CH, CH)
        copies.append(pltpu.async_copy(cwf_hbm.at[co_v.at[sl]], v_f.at[sl], sem))
        copies.append(pltpu.async_copy(xwf_hbm.at[xo_v.at[sl]], u_f.at[sl], sem))
    for cp in copies:
        cp.wait()

    def chunk_body(k, carry):
        acc = jnp.zeros((16,), jnp.float32)
        for d in range(D):
            sl = pl.ds(d * BPW + k * 16, 16)
            acc = acc + v_f[sl] * u_f[sl]
        o_v[pl.ds(k * 16, 16)] = acc
        return carry

    lax.fori_loop(0, BPW // 16, chunk_body, 0)

    pltpu.sync_copy(o_v, out_hbm.at[pl.ds(base, BPW)])


def kernel(tc_center, tc_context, center_weight, context_weight):
    cwf = jnp.transpose(center_weight).reshape(D * NV)
    xwf = jnp.transpose(context_weight).reshape(D * NV)
    return _skipgram_sc(tc_center, tc_context, cwf, xwf)


# speedup vs baseline: 1.0014x; 1.0014x over previous
import functools

import jax
import jax.numpy as jnp
from jax import lax
from jax.experimental import pallas as pl
from jax.experimental.pallas import tpu as pltpu
from jax.experimental.pallas import tpu_sc as plsc

D = 16
NV = 1000000
B = 16384
NC = 2
NS = 16
NW = NC * NS
BPW = B // NW
CH = 128
NCH = BPW // CH

_mesh = plsc.VectorSubcoreMesh(core_axis_name="c", subcore_axis_name="s")


@functools.partial(
    pl.kernel,
    out_type=jax.ShapeDtypeStruct((B,), jnp.float32),
    mesh=_mesh,
    compiler_params=pltpu.CompilerParams(
        needs_layout_passes=False, use_tc_tiling_on_sc=False),
    scratch_types=[
        pltpu.VMEM((BPW,), jnp.int32),
        pltpu.VMEM((BPW,), jnp.int32),
        pltpu.VMEM((BPW * D,), jnp.float32),
        pltpu.VMEM((BPW * D,), jnp.float32),
        pltpu.VMEM((BPW,), jnp.float32),
        pltpu.SemaphoreType.DMA,
    ],
)
def _skipgram_sc(ci_hbm, xi_hbm, cwt_hbm, xwt_hbm, out_hbm,
                 ci_v, xi_v, v_f, u_f, o_v, sem):
    wid = lax.axis_index("s") * NC + lax.axis_index("c")
    base = wid * BPW

    pltpu.sync_copy(ci_hbm.at[pl.ds(base, BPW)], ci_v)
    pltpu.sync_copy(xi_hbm.at[pl.ds(base, BPW)], xi_v)

    copies = []
    for d in range(D):
        crow = cwt_hbm.at[d]
        xrow = xwt_hbm.at[d]
        for c in range(NCH):
            isl = pl.ds(c * CH, CH)
            osl = pl.ds(d * BPW + c * CH, CH)
            copies.append(
                pltpu.async_copy(crow.at[ci_v.at[isl]], v_f.at[osl], sem))
            copies.append(
                pltpu.async_copy(xrow.at[xi_v.at[isl]], u_f.at[osl], sem))
    for cp in copies:
        cp.wait()

    def chunk_body(k, carry):
        acc = jnp.zeros((16,), jnp.float32)
        for d in range(D):
            sl = pl.ds(d * BPW + k * 16, 16)
            acc = acc + v_f[sl] * u_f[sl]
        o_v[pl.ds(k * 16, 16)] = acc
        return carry

    lax.fori_loop(0, BPW // 16, chunk_body, 0)

    pltpu.sync_copy(o_v, out_hbm.at[pl.ds(base, BPW)])


def kernel(tc_center, tc_context, center_weight, context_weight):
    cwt = jnp.transpose(center_weight)
    xwt = jnp.transpose(context_weight)
    return _skipgram_sc(tc_center, tc_context, cwt, xwt)


# speedup vs baseline: 4.7859x; 4.7790x over previous
import functools

import jax
import jax.numpy as jnp
from jax import lax
from jax.experimental import pallas as pl
from jax.experimental.pallas import tpu as pltpu
from jax.experimental.pallas import tpu_sc as plsc

D = 16
B = 16384
NC = 2
NS = 16
NW = NC * NS
BPW = B // NW

_mesh = plsc.VectorSubcoreMesh(core_axis_name="c", subcore_axis_name="s")


@functools.partial(
    pl.kernel,
    out_type=jax.ShapeDtypeStruct((B,), jnp.float32),
    mesh=_mesh,
    compiler_params=pltpu.CompilerParams(needs_layout_passes=False),
    scratch_types=[
        pltpu.VMEM((BPW,), jnp.int32),
        pltpu.VMEM((BPW,), jnp.int32),
        pltpu.VMEM((BPW // 2, D), jnp.float32),
        pltpu.VMEM((BPW // 2, D), jnp.float32),
        pltpu.VMEM((BPW,), jnp.float32),
        pltpu.SemaphoreType.DMA,
    ],
)
def _skipgram_sc(ci_hbm, xi_hbm, cw_hbm, xw_hbm, out_hbm,
                 ci_v, xi_v, v_f, u_f, o_v, sem):
    wid = lax.axis_index("s") * NC + lax.axis_index("c")
    base = wid * BPW

    pltpu.sync_copy(ci_hbm.at[pl.ds(base, BPW)], ci_v)
    pltpu.sync_copy(xi_hbm.at[pl.ds(base, BPW)], xi_v)

    lanes = lax.iota(jnp.int32, 16)
    zeros_i = jnp.zeros((16,), jnp.int32)
    HP = BPW // 2

    for h in range(2):
        hbase = h * HP

        def fire_body(k, carry):
            civ = ci_v[pl.ds(hbase + k * 16, 16)]
            xiv = xi_v[pl.ds(hbase + k * 16, 16)]
            for j in range(16):
                ci = jnp.sum(jnp.where(lanes == j, civ, zeros_i))
                xi = jnp.sum(jnp.where(lanes == j, xiv, zeros_i))
                pltpu.async_copy(cw_hbm.at[ci], v_f.at[k * 16 + j], sem)
                pltpu.async_copy(xw_hbm.at[xi], u_f.at[k * 16 + j], sem)
            return carry

        lax.fori_loop(0, HP // 16, fire_body, 0)
        pltpu.make_async_copy(cw_hbm.at[pl.ds(0, HP)], v_f, sem).wait()
        pltpu.make_async_copy(cw_hbm.at[pl.ds(0, HP)], u_f, sem).wait()

        def chunk_body(k, carry):
            prow = k * 16 + lanes
            acc = jnp.zeros((16,), jnp.float32)
            for d in range(D):
                col = jnp.full((16,), d, jnp.int32)
                cv = plsc.load_gather(v_f, [prow, col])
                cu = plsc.load_gather(u_f, [prow, col])
                acc = acc + cv * cu
            o_v[pl.ds(hbase + k * 16, 16)] = acc
            return carry

        lax.fori_loop(0, HP // 16, chunk_body, 0)

    pltpu.sync_copy(o_v, out_hbm.at[pl.ds(base, BPW)])


def kernel(tc_center, tc_context, center_weight, context_weight):
    return _skipgram_sc(tc_center, tc_context, center_weight, context_weight)
